# baseline (speedup 1.0000x reference)
import functools

import jax
import jax.numpy as jnp
from jax import lax
from jax.experimental import pallas as pl
from jax.experimental.pallas import tpu as pltpu
from jax.experimental.pallas import tpu_sc as plsc

_D = 64
_LANES = 16


def _sc_bow_body(idx_hbm, emb_hbm, freq_hbm, out_hbm,
                 idx_v, rows_v, f_v, acc_v, sem, *, l_pad):
    cid = lax.axis_index("c")
    sid = lax.axis_index("s")

    @pl.when(jnp.logical_and(cid == 0, sid == 0))
    def _():
        pltpu.sync_copy(idx_hbm, idx_v)
        emb_cp = pltpu.async_copy(emb_hbm.at[idx_v], rows_v, sem)
        frq_cp = pltpu.async_copy(freq_hbm.at[idx_v], f_v, sem)
        emb_cp.wait()
        frq_cp.wait()

        def body(k, accs):
            wvec = 1.0 / f_v[pl.ds(k * _LANES, _LANES)]
            base = k * _LANES
            for j in range(_LANES):
                w = wvec[j]
                accs = tuple(
                    accs[c] + w * rows_v[base + j, pl.ds(c * _LANES, _LANES)]
                    for c in range(_D // _LANES)
                )
            return accs

        init = tuple(jnp.zeros((_LANES,), jnp.float32)
                     for _ in range(_D // _LANES))
        accs = lax.fori_loop(0, l_pad // _LANES, body, init)
        for c in range(_D // _LANES):
            acc_v[pl.ds(c * _LANES, _LANES)] = accs[c]
        pltpu.sync_copy(acc_v, out_hbm)


def _make_sc_bow(l_pad):
    return functools.partial(
        pl.kernel,
        out_type=jax.ShapeDtypeStruct((_D,), jnp.float32),
        mesh=plsc.VectorSubcoreMesh(core_axis_name="c", subcore_axis_name="s"),
        scratch_types=[
            pltpu.VMEM((l_pad,), jnp.int32),
            pltpu.VMEM((l_pad, _D), jnp.float32),
            pltpu.VMEM((l_pad,), jnp.float32),
            pltpu.VMEM((_D,), jnp.float32),
            pltpu.SemaphoreType.DMA,
        ],
        compiler_params=pltpu.CompilerParams(use_tc_tiling_on_sc=False),
    )(functools.partial(_sc_bow_body, l_pad=l_pad))


def _tc_head_body(bow_ref, w_ref, b_ref, out_ref, *, scale):
    bow = bow_ref[...] * scale
    logits = lax.dot_general(
        bow, w_ref[...], (((1,), (1,)), ((), ())),
        preferred_element_type=jnp.float32) + b_ref[...]
    m = jnp.max(logits, axis=-1, keepdims=True)
    s = logits - m
    lse = jnp.log(jnp.sum(jnp.exp(s), axis=-1, keepdims=True))
    out_ref[...] = s - lse


def kernel(input, emb_tensor, freq, W, b):
    L = input.shape[1]
    l_pad = ((L + _LANES - 1) // _LANES) * _LANES
    idx = jnp.concatenate(
        [input[0], jnp.zeros((l_pad - L,), jnp.int32)])
    bow = _make_sc_bow(l_pad)(idx, emb_tensor, freq)

    scale = 1.0 / (float(L) * 100000.0)
    out = pl.pallas_call(
        functools.partial(_tc_head_body, scale=scale),
        out_shape=jax.ShapeDtypeStruct((1, 2), jnp.float32),
    )(bow.reshape(1, _D), W, b.reshape(1, 2))
    return out


# speedup vs baseline: 2.4668x; 2.4668x over previous
import functools

import jax
import jax.numpy as jnp
from jax import lax
from jax.experimental import pallas as pl
from jax.experimental.pallas import tpu as pltpu
from jax.experimental.pallas import tpu_sc as plsc

_D = 64
_LANES = 16


def _sc_bow_body(idx_hbm, emb3_hbm, freq_hbm, out_hbm,
                 idx_v, tiles_v, f_v, acc_v, sem, *, l_pad):
    cid = lax.axis_index("c")
    sid = lax.axis_index("s")

    @pl.when(jnp.logical_and(cid == 0, sid == 0))
    def _():
        pltpu.sync_copy(idx_hbm, idx_v)
        frq_cp = pltpu.async_copy(freq_hbm.at[idx_v], f_v, sem)

        frq_cp.wait()

        def body(k, accs):
            base = k * _LANES
            ivec = idx_v[pl.ds(base, _LANES)]
            tvec = jax.lax.shift_right_logical(ivec, 3)
            rvec = jax.lax.bitwise_and(ivec, 7)
            wvec = 1.0 / f_v[pl.ds(base, _LANES)]
            cps = []
            for j in range(_LANES):
                cps.append(pltpu.async_copy(
                    emb3_hbm.at[tvec[j]], tiles_v.at[j], sem))
            for cp in cps:
                cp.wait()
            for j in range(_LANES):
                w = wvec[j]
                r = rvec[j]
                accs = tuple(
                    accs[c] + w * tiles_v[j, r, pl.ds(c * _LANES, _LANES)]
                    for c in range(_D // _LANES)
                )
            return accs

        init = tuple(jnp.zeros((_LANES,), jnp.float32)
                     for _ in range(_D // _LANES))
        accs = lax.fori_loop(0, l_pad // _LANES, body, init)
        for c in range(_D // _LANES):
            acc_v[pl.ds(c * _LANES, _LANES)] = accs[c]
        pltpu.sync_copy(acc_v, out_hbm)


def _make_sc_bow(l_pad, v):
    return functools.partial(
        pl.kernel,
        out_type=jax.ShapeDtypeStruct((_D,), jnp.float32),
        mesh=plsc.VectorSubcoreMesh(core_axis_name="c", subcore_axis_name="s"),
        scratch_types=[
            pltpu.VMEM((l_pad,), jnp.int32),
            pltpu.VMEM((_LANES, 8, _D), jnp.float32),
            pltpu.VMEM((l_pad,), jnp.float32),
            pltpu.VMEM((_D,), jnp.float32),
            pltpu.SemaphoreType.DMA,
        ],
        compiler_params=pltpu.CompilerParams(use_tc_tiling_on_sc=True),
    )(functools.partial(_sc_bow_body, l_pad=l_pad))


def _tc_head_body(bow_ref, w_ref, b_ref, out_ref, *, scale):
    bow = bow_ref[...] * scale
    logits = lax.dot_general(
        bow, w_ref[...], (((1,), (1,)), ((), ())),
        preferred_element_type=jnp.float32) + b_ref[...]
    m = jnp.max(logits, axis=-1, keepdims=True)
    s = logits - m
    lse = jnp.log(jnp.sum(jnp.exp(s), axis=-1, keepdims=True))
    out_ref[...] = s - lse


def kernel(input, emb_tensor, freq, W, b):
    L = input.shape[1]
    V = emb_tensor.shape[0]
    l_pad = ((L + _LANES - 1) // _LANES) * _LANES
    idx = jnp.concatenate(
        [input[0], jnp.zeros((l_pad - L,), jnp.int32)])
    emb3 = emb_tensor.reshape(V // 8, 8, _D)
    bow = _make_sc_bow(l_pad, V)(idx, emb3, freq)

    scale = 1.0 / (float(L) * 100000.0)
    out = pl.pallas_call(
        functools.partial(_tc_head_body, scale=scale),
        out_shape=jax.ShapeDtypeStruct((1, 2), jnp.float32),
    )(bow.reshape(1, _D), W, b.reshape(1, 2))
    return out
